# baseline (speedup 1.0000x reference)
import jax
import jax.numpy as jnp
from jax import lax
from jax.experimental import pallas as pl
from jax.experimental.pallas import tpu as pltpu
from jax.experimental.pallas import tpu_sc as plsc

T, H, W = 16, 32, 32
C = 128
B = 8
NTOK = (T // 2) * (H // 2) * (W // 2)
NROWS = NTOK * B * 8

NC, NS, L = 2, 16, 16
NW = NC * NS
EL_PER_W = T * H * W * B // NW
CHUNK = 64
NCHUNK = EL_PER_W // CHUNK
NBUF = 8
LOOKAHEAD = 5
VPC = CHUNK // L

_mesh = plsc.VectorSubcoreMesh(
    core_axis_name="c", subcore_axis_name="s", num_cores=NC, num_subcores=NS
)


def _body(x_hbm, table_hbm, out_hbm, xin, oidx, rows, *sems):
    gsem = sems[:NBUF]
    ssem = sems[NBUF:]
    wid = lax.axis_index("s") * NC + lax.axis_index("c")

    pltpu.sync_copy(x_hbm.at[pl.ds(wid * EL_PER_W, EL_PER_W)], xin)

    def _gather(c, s):
        pltpu.make_async_copy(
            table_hbm.at[xin.at[pl.ds(c * CHUNK, CHUNK)]], rows.at[s], gsem[s]
        ).start()

    for s in range(LOOKAHEAD):
        _gather(s, s)

    iota = lax.iota(jnp.int32, L)
    mbase = wid * EL_PER_W

    @pl.loop(0, EL_PER_W // L)
    def _compute_oidx(j0):
        m = mbase + j0 * L + iota
        b = m & 7
        i = m >> 3
        w_ = i & 31
        h = (i >> 5) & 31
        t = i >> 10
        j = (
            (t >> 1) * 16384
            + (h >> 1) * 1024
            + (w_ >> 1) * 64
            + (w_ & 1) * 32
            + (h & 1) * 16
            + (t & 1) * 8
            + b
        )
        oidx[j0 // VPC, pl.ds((j0 % VPC) * L, L)] = j

    def _scatter(c, s):
        pltpu.make_async_copy(rows.at[s], out_hbm.at[oidx.at[c]], ssem[s]).start()

    def _wait_gather(c, s):
        pltpu.make_async_copy(
            table_hbm.at[xin.at[pl.ds(c * CHUNK, CHUNK)]], rows.at[s], gsem[s]
        ).wait()

    def _wait_scatter(c, s):
        pltpu.make_async_copy(rows.at[s], out_hbm.at[oidx.at[c]], ssem[s]).wait()

    @pl.loop(0, NCHUNK, step=NBUF)
    def _chunk_loop(c0):
        for s in range(NBUF):
            c = c0 + s
            _wait_gather(c, s)
            _scatter(c, s)
            s2 = (s + LOOKAHEAD) % NBUF
            cn = c + LOOKAHEAD

            @pl.when(cn >= NBUF)
            def _():
                _wait_scatter(cn - NBUF, s2)

            @pl.when(cn < NCHUNK)
            def _():
                _gather(cn, s2)

    for c in range(NCHUNK - NBUF + LOOKAHEAD, NCHUNK):
        _wait_scatter(c, c % NBUF)


_lookup = pl.kernel(
    _body,
    out_type=jax.ShapeDtypeStruct((NROWS, C), jnp.float32),
    mesh=_mesh,
    scratch_types=[
        pltpu.VMEM((EL_PER_W,), jnp.int32),
        pltpu.VMEM((NCHUNK, CHUNK), jnp.int32),
        pltpu.VMEM((NBUF, CHUNK, C), jnp.float32),
    ]
    + [pltpu.SemaphoreType.DMA] * (2 * NBUF),
)


@jax.jit
def kernel(x, table):
    out = _lookup(x.reshape(-1), table)
    e = out.reshape(NTOK, 8, B, C)
    return jnp.swapaxes(e, 1, 2).reshape(NTOK, B, 8 * C)


# speedup vs baseline: 1.0074x; 1.0074x over previous
import jax
import jax.numpy as jnp
from jax import lax
from jax.experimental import pallas as pl
from jax.experimental.pallas import tpu as pltpu
from jax.experimental.pallas import tpu_sc as plsc

T, H, W = 16, 32, 32
C = 128
B = 8
NTOK = (T // 2) * (H // 2) * (W // 2)
NROWS = NTOK * B * 8

NC, NS, L = 2, 16, 16
NW = NC * NS
EL_PER_W = T * H * W * B // NW
CHUNK = 64
NCHUNK = EL_PER_W // CHUNK
NBUF = 8
LOOKAHEAD = 7
VPC = CHUNK // L

_mesh = plsc.VectorSubcoreMesh(
    core_axis_name="c", subcore_axis_name="s", num_cores=NC, num_subcores=NS
)


def _body(x_hbm, table_hbm, out_hbm, xin, oidx, rows, *sems):
    gsem = sems[:NBUF]
    ssem = sems[NBUF:]
    wid = lax.axis_index("s") * NC + lax.axis_index("c")

    pltpu.sync_copy(x_hbm.at[pl.ds(wid * EL_PER_W, EL_PER_W)], xin)

    def _gather(c, s):
        pltpu.make_async_copy(
            table_hbm.at[xin.at[pl.ds(c * CHUNK, CHUNK)]], rows.at[s], gsem[s]
        ).start()

    for s in range(LOOKAHEAD):
        _gather(s, s)

    iota = lax.iota(jnp.int32, L)
    mbase = wid * EL_PER_W

    @pl.loop(0, EL_PER_W // L)
    def _compute_oidx(j0):
        m = mbase + j0 * L + iota
        b = m & 7
        i = m >> 3
        w_ = i & 31
        h = (i >> 5) & 31
        t = i >> 10
        j = (
            (t >> 1) * 16384
            + (h >> 1) * 1024
            + (w_ >> 1) * 64
            + (w_ & 1) * 32
            + (h & 1) * 16
            + (t & 1) * 8
            + b
        )
        oidx[j0 // VPC, pl.ds((j0 % VPC) * L, L)] = j

    def _scatter(c, s):
        pltpu.make_async_copy(rows.at[s], out_hbm.at[oidx.at[c]], ssem[s]).start()

    def _wait_gather(c, s):
        pltpu.make_async_copy(
            table_hbm.at[xin.at[pl.ds(c * CHUNK, CHUNK)]], rows.at[s], gsem[s]
        ).wait()

    def _wait_scatter(c, s):
        pltpu.make_async_copy(rows.at[s], out_hbm.at[oidx.at[c]], ssem[s]).wait()

    @pl.loop(0, NCHUNK, step=NBUF)
    def _chunk_loop(c0):
        for s in range(NBUF):
            c = c0 + s
            _wait_gather(c, s)
            _scatter(c, s)
            s2 = (s + LOOKAHEAD) % NBUF
            cn = c + LOOKAHEAD

            @pl.when(cn >= NBUF)
            def _():
                _wait_scatter(cn - NBUF, s2)

            @pl.when(cn < NCHUNK)
            def _():
                _gather(cn, s2)

    for c in range(NCHUNK - NBUF + LOOKAHEAD, NCHUNK):
        _wait_scatter(c, c % NBUF)


_lookup = pl.kernel(
    _body,
    out_type=jax.ShapeDtypeStruct((NROWS, C), jnp.float32),
    mesh=_mesh,
    scratch_types=[
        pltpu.VMEM((EL_PER_W,), jnp.int32),
        pltpu.VMEM((NCHUNK, CHUNK), jnp.int32),
        pltpu.VMEM((NBUF, CHUNK, C), jnp.float32),
    ]
    + [pltpu.SemaphoreType.DMA] * (2 * NBUF),
)


@jax.jit
def kernel(x, table):
    out = _lookup(x.reshape(-1), table)
    e = out.reshape(NTOK, 8, B, C)
    return jnp.swapaxes(e, 1, 2).reshape(NTOK, B, 8 * C)
